# Initial kernel scaffold; baseline (speedup 1.0000x reference)
import jax
import jax.numpy as jnp
from jax.experimental import pallas as pl


def kernel(feature, dists_max, dists_argmax, dc_w1, dc_b1, dc_w2, dc_b2, wh, bh, wo, bo):
    raise NotImplementedError("write your pallas kernel here")


# speedup vs baseline: 2.1203x; 2.1203x over previous
import functools

import jax
import jax.numpy as jnp
from jax import lax
from jax.experimental import pallas as pl
from jax.experimental.pallas import tpu as pltpu
from jax.experimental.pallas import tpu_sc as plsc

_N, _K, _IN, _OUT = 10000, 32, 128, 128
_NC, _NS, _L = 2, 16, 16
_NW = _NC * _NS
_NP = 10240
_NODES_W = _NP // _NW
_C = 4
_CK = _C * _K
_CHUNKS = _NODES_W // _C
_NCH = _NP // _C
_R = 1 + _C
_BT = 256
_V = _OUT // _L


def _tc_precompute(feat_ref, dmaxt_ref, w1t_ref, w2t_ref, bh_ref,
                   w1c_ref, b1c_ref, w2c_ref, b2c_ref,
                   g_ref, s_ref, distst_ref):
    f = feat_ref[...]
    g_ref[...] = jnp.dot(f, w1t_ref[...], preferred_element_type=jnp.float32)
    s_ref[...] = jnp.dot(f, w2t_ref[...],
                         preferred_element_type=jnp.float32) + bh_ref[...]
    x = dmaxt_ref[...]
    acc = jnp.zeros_like(x) + b2c_ref[0, 0]
    for j in range(_OUT):
        acc = acc + jnp.maximum(x * w1c_ref[0, j] + b1c_ref[0, j],
                                0.0) * w2c_ref[0, j]
    distst_ref[...] = acc


@functools.cache
def _build_sc_gather_combine():
    mesh = plsc.VectorSubcoreMesh(core_axis_name="c", subcore_axis_name="s")
    return functools.partial(
        pl.kernel,
        mesh=mesh,
        out_type=jax.ShapeDtypeStruct((_NCH * _R * _OUT,), jnp.float32),
        scratch_types=[
            pltpu.VMEM((2, _CK), jnp.int32),
            pltpu.VMEM((2, _CK, _OUT), jnp.float32),
            pltpu.VMEM((2, _R * _OUT), jnp.float32),
            pltpu.VMEM((2, _R * _OUT), jnp.float32),
            pltpu.VMEM((_OUT,), jnp.float32),
            pltpu.VMEM((_L,), jnp.float32),
            pltpu.SemaphoreType.DMA,
            pltpu.SemaphoreType.DMA,
            pltpu.SemaphoreType.DMA,
            pltpu.SemaphoreType.DMA,
            pltpu.SemaphoreType.DMA,
            pltpu.SemaphoreType.DMA,
            pltpu.SemaphoreType.DMA,
            pltpu.SemaphoreType.DMA,
            pltpu.SemaphoreType.DMA,
        ],
    )(_sc_gather_combine_body)


def _gather_lanes(vec, idx):
    return lax.gather(
        vec, idx[:, None],
        lax.GatherDimensionNumbers(offset_dims=(), collapsed_slice_dims=(0,),
                                   start_index_map=(0,)),
        (1,), mode=lax.GatherScatterMode.PROMISE_IN_BOUNDS)


def _bcast_lane(vec, j):
    return _gather_lanes(vec, jnp.full((_L,), j, jnp.int32))


def _sc_gather_combine_body(g_hbm, ds_hbm, idx_hbm, wo_hbm, bo_hbm,
                            out_hbm,
                            idx_v, rows_v, ds_v, out_v, wo_v, bo_v,
                            sem_i0, sem_i1, sem_d0, sem_d1,
                            sem_g0, sem_g1, sem_w0, sem_w1, sem_p):
    wid = lax.axis_index("s") * _NC + lax.axis_index("c")
    chunk0 = wid * _CHUNKS
    sem_i = (sem_i0, sem_i1)
    sem_d = (sem_d0, sem_d1)
    sem_g = (sem_g0, sem_g1)
    sem_w = (sem_w0, sem_w1)

    pltpu.async_copy(wo_hbm, wo_v, sem_p)
    pltpu.make_async_copy(wo_hbm, wo_v, sem_p).wait()
    pltpu.async_copy(bo_hbm, bo_v, sem_p)
    pltpu.make_async_copy(bo_hbm, bo_v, sem_p).wait()
    wo_regs = [wo_v[pl.ds(_L * v, _L)] for v in range(_V)]
    bo_s = bo_v[...][0]
    inv_k = jnp.float32(1.0 / _K)
    lane = lax.iota(jnp.int32, _L)
    xor_idx = [lane ^ sh for sh in (8, 4, 2, 1)]

    def start_idx(g, b):
        pltpu.async_copy(idx_hbm.at[pl.ds((chunk0 + g) * _CK, _CK)],
                         idx_v.at[b], sem_i[b])

    def wait_idx(b):
        pltpu.make_async_copy(idx_hbm.at[pl.ds(0, _CK)], idx_v.at[b],
                              sem_i[b]).wait()

    def start_ds(g, b):
        pltpu.async_copy(ds_hbm.at[pl.ds((chunk0 + g) * _R * _OUT, _R * _OUT)],
                         ds_v.at[b], sem_d[b])

    def wait_ds(b):
        pltpu.make_async_copy(ds_hbm.at[pl.ds(0, _R * _OUT)], ds_v.at[b],
                              sem_d[b]).wait()

    def start_gather(b):
        pltpu.async_copy(g_hbm.at[idx_v.at[b]], rows_v.at[b], sem_g[b])

    def wait_gather(b):
        pltpu.make_async_copy(g_hbm.at[idx_v.at[b]], rows_v.at[b],
                              sem_g[b]).wait()

    def start_wb(g, b):
        pltpu.async_copy(out_v.at[b],
                         out_hbm.at[pl.ds((chunk0 + g) * _R * _OUT, _R * _OUT)],
                         sem_w[b])

    def wait_wb(b):
        pltpu.make_async_copy(out_v.at[b], out_hbm.at[pl.ds(0, _R * _OUT)],
                              sem_w[b]).wait()

    def compute(b):
        def node_body(c, carry2):
            s_regs = [ds_v[b, pl.ds((1 + c) * _OUT + _L * v, _L)]
                      for v in range(_V)]
            accs = tuple(jnp.zeros((_L,), jnp.float32) for _ in range(_V))
            for hh in range(_K // _L):
                hbase = c * _K + hh * _L
                dvec = ds_v[b, pl.ds(hbase, _L)]

                def k_body(j, carry3, hbase=hbase, dvec=dvec):
                    accs3, pos_acc = carry3
                    i = hbase + j
                    db = _bcast_lane(dvec, j)
                    out = []
                    pos = None
                    for v in range(_V):
                        row = rows_v[b, i, pl.ds(_L * v, _L)]
                        m = jnp.maximum(db * row + s_regs[v], 0.0)
                        out.append(accs3[v] + m)
                        pv = m * wo_regs[v]
                        pos = pv if pos is None else pos + pv
                    for xi in xor_idx:
                        pos = pos + _gather_lanes(pos, xi)
                    pos_acc = jnp.where(lane == j, pos, pos_acc)
                    return tuple(out), pos_acc

                accs, pos_acc = lax.fori_loop(
                    0, _L, k_body, (accs, jnp.zeros((_L,), jnp.float32)))
                out_v[b, pl.ds(hbase, _L)] = pos_acc + bo_s
            for v in range(_V):
                out_v[b, pl.ds((1 + c) * _OUT + _L * v, _L)] = accs[v] * inv_k
            return carry2

        lax.fori_loop(0, _C, node_body, 0)

    start_idx(0, 0)
    start_ds(0, 0)
    start_idx(1, 1)
    start_ds(1, 1)
    wait_idx(0)
    start_gather(0)

    def pair_body(gp, carry):
        for b in range(2):
            g = gp * 2 + b
            o = 1 - b
            wait_gather(b)

            @pl.when(g + 2 < _CHUNKS)
            def _():
                start_idx(g + 2, b)

            @pl.when(g + 1 < _CHUNKS)
            def _():
                wait_idx(o)
                start_gather(o)

            wait_ds(b)

            @pl.when(g >= 2)
            def _():
                wait_wb(b)

            compute(b)
            start_wb(g, b)

            @pl.when(g + 2 < _CHUNKS)
            def _():
                start_ds(g + 2, b)
        return carry

    lax.fori_loop(0, _CHUNKS // 2, pair_body, 0)
    wait_wb(0)
    wait_wb(1)


def kernel(feature, dists_max, dists_argmax, dc_w1, dc_b1, dc_w2, dc_b2,
           wh, bh, wo, bo):
    pad = _NP - _N
    feat_p = jnp.pad(feature.astype(jnp.float32), ((0, pad), (0, 0)))
    dmax_p = jnp.pad(dists_max.astype(jnp.float32), ((0, pad), (0, 0)))
    idx_p = jnp.pad(dists_argmax.astype(jnp.int32), ((0, pad), (0, 0)))

    w1t = wh[:, :_IN].T
    w2t = wh[:, _IN:].T
    bh2 = bh.reshape(1, _OUT)
    w1c = dc_w1.reshape(1, _OUT)
    b1c = dc_b1.reshape(1, _OUT)
    w2c = dc_w2.reshape(1, _OUT)
    b2c = dc_b2.reshape(1, 1)

    g, s, distst = pl.pallas_call(
        _tc_precompute,
        grid=(_NP // _BT,),
        in_specs=[
            pl.BlockSpec((_BT, _IN), lambda i: (i, 0)),
            pl.BlockSpec((_K, _BT), lambda i: (0, i)),
            pl.BlockSpec((_IN, _OUT), lambda i: (0, 0)),
            pl.BlockSpec((_IN, _OUT), lambda i: (0, 0)),
            pl.BlockSpec((1, _OUT), lambda i: (0, 0)),
            pl.BlockSpec((1, _OUT), lambda i: (0, 0)),
            pl.BlockSpec((1, _OUT), lambda i: (0, 0)),
            pl.BlockSpec((1, _OUT), lambda i: (0, 0)),
            pl.BlockSpec((1, 1), lambda i: (0, 0)),
        ],
        out_specs=[
            pl.BlockSpec((_BT, _OUT), lambda i: (i, 0)),
            pl.BlockSpec((_BT, _OUT), lambda i: (i, 0)),
            pl.BlockSpec((_K, _BT), lambda i: (0, i)),
        ],
        out_shape=[
            jax.ShapeDtypeStruct((_NP, _OUT), jnp.float32),
            jax.ShapeDtypeStruct((_NP, _OUT), jnp.float32),
            jax.ShapeDtypeStruct((_K, _NP), jnp.float32),
        ],
    )(feat_p, dmax_p.T, w1t, w2t, bh2, w1c, b1c, w2c, b2c)

    dists = distst.T
    ds_comb = jnp.concatenate(
        [dists.reshape(_NCH, 1, _OUT), s.reshape(_NCH, _C, _OUT)],
        axis=1).reshape(_NCH * _R * _OUT)

    out_comb = _build_sc_gather_combine()(
        g, ds_comb, idx_p.reshape(-1), wo.reshape(-1),
        jnp.pad(bo, (0, _L - 1)))

    oc = out_comb.reshape(_NCH, _R, _OUT)
    out_position = oc[:, 0, :].reshape(_NP, _K)[:_N]
    out_structure = oc[:, 1:, :].reshape(_NP, _OUT)[:_N]
    return out_position, out_structure
